# Initial kernel scaffold; baseline (speedup 1.0000x reference)
import jax
import jax.numpy as jnp
from jax.experimental import pallas as pl


def kernel(x, edge_index, emb, W1, att_src1, att_dst1, b1, W2, att_src2, att_dst2, b2):
    raise NotImplementedError("write your pallas kernel here")


# speedup vs baseline: 3.6501x; 3.6501x over previous
import functools
import jax
import jax.numpy as jnp
import numpy as np
from jax import lax
from jax.experimental import pallas as pl
from jax.experimental.pallas import tpu as pltpu
from jax.experimental.pallas import tpu_sc as plsc

_INFO = plsc.get_sparse_core_info()
_NC, _NS, _L = _INFO.num_cores, _INFO.num_subcores, _INFO.num_lanes
_NW = _NC * _NS


@functools.partial(jax.jit, static_argnames=("b_pad",))
def _sc_gather_rows(table, idx, b_pad):
    d = table.shape[1]
    assert d == 128
    b_per_w = b_pad // _NW
    n_chunk = 2
    ch = b_per_w // n_chunk
    mesh = plsc.VectorSubcoreMesh(core_axis_name="c", subcore_axis_name="s")

    @functools.partial(
        pl.kernel,
        mesh=mesh,
        out_type=jax.ShapeDtypeStruct((b_pad, d), jnp.float32),
        scratch_types=[
            pltpu.VMEM((ch,), jnp.int32),
            pltpu.VMEM((ch, d), jnp.float32),
            pltpu.SemaphoreType.DMA,
        ],
    )
    def k(table_hbm, idx_hbm, out_hbm, idx_v, rows_v, sem):
        wid = lax.axis_index("s") * _NC + lax.axis_index("c")
        for c in range(n_chunk):
            base = wid * b_per_w + c * ch
            pltpu.sync_copy(idx_hbm.at[pl.ds(base, ch)], idx_v)
            pltpu.async_copy(table_hbm.at[idx_v], rows_v, sem).wait()
            pltpu.sync_copy(rows_v, out_hbm.at[pl.ds(base, ch)])

    return k(table, idx)


def _tc_dense(f, W, att_src_flat, att_dst_flat, heads, b_prev=None):
    n, d = f.shape
    bn = 1000
    grid = (n // bn,)

    def body(f_ref, w_ref, asrc_ref, adst_ref, bprev_ref, h_ref, as_ref, ad_ref):
        x = f_ref[...]
        if b_prev is not None:
            x = jnp.maximum(x + bprev_ref[...], 0.0)
        h = jnp.dot(x, w_ref[...], preferred_element_type=jnp.float32)
        h_ref[...] = h
        ps = h * asrc_ref[...]
        pd = h * adst_ref[...]
        ch = h.shape[1] // heads
        acc_s = []
        acc_d = []
        for hh in range(heads):
            acc_s.append(ps[:, hh * ch:(hh + 1) * ch].sum(axis=1, keepdims=True))
            acc_d.append(pd[:, hh * ch:(hh + 1) * ch].sum(axis=1, keepdims=True))
        pad = jnp.zeros((x.shape[0], 16 - heads), jnp.float32)
        as_ref[...] = jnp.concatenate(acc_s + [pad], axis=1)
        ad_ref[...] = jnp.concatenate(acc_d + [pad], axis=1)

    bprev = b_prev if b_prev is not None else jnp.zeros((1, d), jnp.float32)
    h, a_s, a_d = pl.pallas_call(
        body,
        grid=grid,
        in_specs=[
            pl.BlockSpec((bn, d), lambda i: (i, 0)),
            pl.BlockSpec((d, d), lambda i: (0, 0)),
            pl.BlockSpec((1, d), lambda i: (0, 0)),
            pl.BlockSpec((1, d), lambda i: (0, 0)),
            pl.BlockSpec((1, d), lambda i: (0, 0)),
        ],
        out_specs=[
            pl.BlockSpec((bn, d), lambda i: (i, 0)),
            pl.BlockSpec((bn, 16), lambda i: (i, 0)),
            pl.BlockSpec((bn, 16), lambda i: (i, 0)),
        ],
        out_shape=[
            jax.ShapeDtypeStruct((n, d), jnp.float32),
            jax.ShapeDtypeStruct((n, 16), jnp.float32),
            jax.ShapeDtypeStruct((n, 16), jnp.float32),
        ],
    )(f, W, att_src_flat.reshape(1, d), att_dst_flat.reshape(1, d), bprev)
    return h, a_s, a_d


def _edge_aggregate(h, a_s, a_d, src, dst, heads, num_nodes):
    alpha = a_s[src, :heads] + a_d[dst, :heads]
    alpha = jnp.where(alpha >= 0, alpha, 0.2 * alpha)
    amax = jax.ops.segment_max(alpha, dst, num_segments=num_nodes)
    amax = jnp.where(jnp.isfinite(amax), amax, 0.0)
    ex = jnp.exp(alpha - amax[dst])
    denom = jax.ops.segment_sum(ex, dst, num_segments=num_nodes)
    coef = ex / (denom[dst] + 1e-16)
    ch = h.shape[1] // heads
    coef_full = jnp.repeat(coef, ch, axis=1)
    msg = h[src] * coef_full
    return jax.ops.segment_sum(msg, dst, num_segments=num_nodes)


def kernel(x, edge_index, emb, W1, att_src1, att_dst1, b1, W2, att_src2,
           att_dst2, b2):
    n = x.shape[0]
    d = emb.shape[1]
    b_pad = ((n + 16 * _NW - 1) // (16 * _NW)) * (16 * _NW)
    idx = jnp.zeros((b_pad,), jnp.int32).at[:n].set(x.astype(jnp.int32))
    emb_pad = jnp.pad(emb, ((0, 0), (0, 128 - d)))
    h0 = _sc_gather_rows(emb_pad, idx, b_pad)[:n, :d]

    loop = jnp.arange(n, dtype=edge_index.dtype)
    src = jnp.concatenate([edge_index[0], loop])
    dst = jnp.concatenate([edge_index[1], loop])

    h1, as1, ad1 = _tc_dense(h0, W1, att_src1.reshape(-1), att_dst1.reshape(-1), 2)
    out1 = _edge_aggregate(h1, as1, ad1, src, dst, 2, n)

    h2, as2, ad2 = _tc_dense(out1, W2, att_src2.reshape(-1), att_dst2.reshape(-1),
                             1, b_prev=b1.reshape(1, d))
    out2 = _edge_aggregate(h2, as2, ad2, src, dst, 1, n)
    return out2 + b2.reshape(1, d)


# speedup vs baseline: 3.7416x; 1.0251x over previous
import functools
import jax
import jax.numpy as jnp
import numpy as np
from jax import lax
from jax.experimental import pallas as pl
from jax.experimental.pallas import tpu as pltpu
from jax.experimental.pallas import tpu_sc as plsc

_INFO = plsc.get_sparse_core_info()
_NC, _NS, _L = _INFO.num_cores, _INFO.num_subcores, _INFO.num_lanes
_NW = _NC * _NS


@functools.partial(jax.jit, static_argnames=("b_pad", "ch"))
def _sc_gather_rows(table, idx, b_pad, ch):
    d = table.shape[1]
    assert d == 128
    b_per_w = b_pad // _NW
    n_chunk = b_per_w // ch
    assert n_chunk * ch == b_per_w and ch % 8 == 0
    mesh = plsc.VectorSubcoreMesh(core_axis_name="c", subcore_axis_name="s")

    @functools.partial(
        pl.kernel,
        mesh=mesh,
        out_type=jax.ShapeDtypeStruct((b_pad, d), jnp.float32),
        scratch_types=[
            pltpu.VMEM((ch,), jnp.int32),
            pltpu.VMEM((ch, d), jnp.float32),
            pltpu.SemaphoreType.DMA,
        ],
    )
    def k(table_hbm, idx_hbm, out_hbm, idx_v, rows_v, sem):
        wid = lax.axis_index("s") * _NC + lax.axis_index("c")
        for c in range(n_chunk):
            base = wid * b_per_w + c * ch
            pltpu.sync_copy(idx_hbm.at[pl.ds(base, ch)], idx_v)
            pltpu.async_copy(table_hbm.at[idx_v], rows_v, sem).wait()
            pltpu.sync_copy(rows_v, out_hbm.at[pl.ds(base, ch)])

    return k(table, idx)


def _tc_dense(f, W, att_src_flat, att_dst_flat, heads, b_prev=None):
    n, d = f.shape
    bn = 1000
    grid = (n // bn,)

    def body(f_ref, w_ref, asrc_ref, adst_ref, bprev_ref, h_ref, as_ref, ad_ref):
        x = f_ref[...]
        if b_prev is not None:
            x = jnp.maximum(x + bprev_ref[...], 0.0)
        h = jnp.dot(x, w_ref[...], preferred_element_type=jnp.float32)
        h_ref[...] = h
        ps = h * asrc_ref[...]
        pd = h * adst_ref[...]
        ch = h.shape[1] // heads
        acc_s = []
        acc_d = []
        for hh in range(heads):
            acc_s.append(ps[:, hh * ch:(hh + 1) * ch].sum(axis=1, keepdims=True))
            acc_d.append(pd[:, hh * ch:(hh + 1) * ch].sum(axis=1, keepdims=True))
        pad = jnp.zeros((x.shape[0], 16 - heads), jnp.float32)
        as_ref[...] = jnp.concatenate(acc_s + [pad], axis=1)
        ad_ref[...] = jnp.concatenate(acc_d + [pad], axis=1)

    bprev = b_prev if b_prev is not None else jnp.zeros((1, d), jnp.float32)
    h, a_s, a_d = pl.pallas_call(
        body,
        grid=grid,
        in_specs=[
            pl.BlockSpec((bn, d), lambda i: (i, 0)),
            pl.BlockSpec((d, d), lambda i: (0, 0)),
            pl.BlockSpec((1, d), lambda i: (0, 0)),
            pl.BlockSpec((1, d), lambda i: (0, 0)),
            pl.BlockSpec((1, d), lambda i: (0, 0)),
        ],
        out_specs=[
            pl.BlockSpec((bn, d), lambda i: (i, 0)),
            pl.BlockSpec((bn, 16), lambda i: (i, 0)),
            pl.BlockSpec((bn, 16), lambda i: (i, 0)),
        ],
        out_shape=[
            jax.ShapeDtypeStruct((n, d), jnp.float32),
            jax.ShapeDtypeStruct((n, 16), jnp.float32),
            jax.ShapeDtypeStruct((n, 16), jnp.float32),
        ],
    )(f, W, att_src_flat.reshape(1, d), att_dst_flat.reshape(1, d), bprev)
    return h, a_s, a_d


def _edge_aggregate(srcrows, dstrows, src, dst, heads, num_nodes):
    a_s_src = srcrows[:, :heads]
    a_d_dst = dstrows[:, 16:16 + heads]
    h_src = srcrows[:, 32:96]
    alpha = a_s_src + a_d_dst
    alpha = jnp.where(alpha >= 0, alpha, 0.2 * alpha)
    amax = jax.ops.segment_max(alpha, dst, num_segments=num_nodes)
    amax = jnp.where(jnp.isfinite(amax), amax, 0.0)
    ex = jnp.exp(alpha - amax[dst])
    denom = jax.ops.segment_sum(ex, dst, num_segments=num_nodes)
    coef = ex / (denom[dst] + 1e-16)
    ch = h_src.shape[1] // heads
    coef_full = jnp.repeat(coef, ch, axis=1)
    msg = h_src * coef_full
    return jax.ops.segment_sum(msg, dst, num_segments=num_nodes)


def kernel(x, edge_index, emb, W1, att_src1, att_dst1, b1, W2, att_src2,
           att_dst2, b2):
    n = x.shape[0]
    d = emb.shape[1]
    b_pad = ((n + 16 * _NW - 1) // (16 * _NW)) * (16 * _NW)
    idx = jnp.zeros((b_pad,), jnp.int32).at[:n].set(x.astype(jnp.int32))
    emb_pad = jnp.pad(emb, ((0, 0), (0, 128 - d)))
    h0 = _sc_gather_rows(emb_pad, idx, b_pad, b_pad // _NW // 2)[:n, :d]

    loop = jnp.arange(n, dtype=edge_index.dtype)
    src = jnp.concatenate([edge_index[0], loop])
    dst = jnp.concatenate([edge_index[1], loop])
    en = src.shape[0]
    e_pad = ((en + 512 * _NW - 1) // (512 * _NW)) * (512 * _NW)
    src_p = jnp.zeros((e_pad,), jnp.int32).at[:en].set(src.astype(jnp.int32))
    dst_p = jnp.zeros((e_pad,), jnp.int32).at[:en].set(dst.astype(jnp.int32))
    zpad = jnp.zeros((n, 32), jnp.float32)

    def layer(f, W, att_s, att_d, heads, b_prev):
        h, a_s, a_d = _tc_dense(f, W, att_s.reshape(-1), att_d.reshape(-1),
                                heads, b_prev=b_prev)
        packed = jnp.concatenate([a_s, a_d, h, zpad], axis=1)
        srows = _sc_gather_rows(packed, src_p, e_pad, 512)[:en]
        drows = _sc_gather_rows(packed, dst_p, e_pad, 512)[:en]
        return _edge_aggregate(srows, drows, src, dst, heads, n)

    out1 = layer(h0, W1, att_src1, att_dst1, 2, None)
    out2 = layer(out1, W2, att_src2, att_dst2, 1, b1.reshape(1, d))
    return out2 + b2.reshape(1, d)


# speedup vs baseline: 5.8642x; 1.5673x over previous
import functools
import jax
import jax.numpy as jnp
import numpy as np
from jax import lax
from jax.experimental import pallas as pl
from jax.experimental.pallas import tpu as pltpu
from jax.experimental.pallas import tpu_sc as plsc

_INFO = plsc.get_sparse_core_info()
_NC, _NS, _L = _INFO.num_cores, _INFO.num_subcores, _INFO.num_lanes
_NW = _NC * _NS


@functools.partial(jax.jit, static_argnames=("b_pad", "ch"))
def _sc_gather_rows(table, idx, b_pad, ch):
    d = table.shape[1]
    assert d == 128
    b_per_w = b_pad // _NW
    n_chunk = b_per_w // ch
    assert n_chunk * ch == b_per_w and ch % 8 == 0
    mesh = plsc.VectorSubcoreMesh(core_axis_name="c", subcore_axis_name="s")

    @functools.partial(
        pl.kernel,
        mesh=mesh,
        out_type=jax.ShapeDtypeStruct((b_pad, d), jnp.float32),
        scratch_types=[
            pltpu.VMEM((ch,), jnp.int32),
            pltpu.VMEM((ch, d), jnp.float32),
            pltpu.SemaphoreType.DMA,
        ],
    )
    def k(table_hbm, idx_hbm, out_hbm, idx_v, rows_v, sem):
        wid = lax.axis_index("s") * _NC + lax.axis_index("c")
        for c in range(n_chunk):
            base = wid * b_per_w + c * ch
            pltpu.sync_copy(idx_hbm.at[pl.ds(base, ch)], idx_v)
            pltpu.async_copy(table_hbm.at[idx_v], rows_v, sem).wait()
            pltpu.sync_copy(rows_v, out_hbm.at[pl.ds(base, ch)])

    return k(table, idx)


def _tc_dense(f, W, att_src_flat, att_dst_flat, heads, b_prev=None):
    n, d = f.shape
    bn = 1000
    grid = (n // bn,)

    def body(f_ref, w_ref, asrc_ref, adst_ref, bprev_ref, h_ref, as_ref, ad_ref):
        x = f_ref[...]
        if b_prev is not None:
            x = jnp.maximum(x + bprev_ref[...], 0.0)
        h = jnp.dot(x, w_ref[...], preferred_element_type=jnp.float32)
        h_ref[...] = h
        ps = h * asrc_ref[...]
        pd = h * adst_ref[...]
        ch = h.shape[1] // heads
        acc_s = []
        acc_d = []
        for hh in range(heads):
            acc_s.append(ps[:, hh * ch:(hh + 1) * ch].sum(axis=1, keepdims=True))
            acc_d.append(pd[:, hh * ch:(hh + 1) * ch].sum(axis=1, keepdims=True))
        pad = jnp.zeros((x.shape[0], 16 - heads), jnp.float32)
        as_ref[...] = jnp.concatenate(acc_s + [pad], axis=1)
        ad_ref[...] = jnp.concatenate(acc_d + [pad], axis=1)

    bprev = b_prev if b_prev is not None else jnp.zeros((1, d), jnp.float32)
    h, a_s, a_d = pl.pallas_call(
        body,
        grid=grid,
        in_specs=[
            pl.BlockSpec((bn, d), lambda i: (i, 0)),
            pl.BlockSpec((d, d), lambda i: (0, 0)),
            pl.BlockSpec((1, d), lambda i: (0, 0)),
            pl.BlockSpec((1, d), lambda i: (0, 0)),
            pl.BlockSpec((1, d), lambda i: (0, 0)),
        ],
        out_specs=[
            pl.BlockSpec((bn, d), lambda i: (i, 0)),
            pl.BlockSpec((bn, 16), lambda i: (i, 0)),
            pl.BlockSpec((bn, 16), lambda i: (i, 0)),
        ],
        out_shape=[
            jax.ShapeDtypeStruct((n, d), jnp.float32),
            jax.ShapeDtypeStruct((n, 16), jnp.float32),
            jax.ShapeDtypeStruct((n, 16), jnp.float32),
        ],
    )(f, W, att_src_flat.reshape(1, d), att_dst_flat.reshape(1, d), bprev)
    return h, a_s, a_d


def _edge_aggregate(srcrows, dstrows, src, dst, heads, num_nodes):
    a_s_src = srcrows[:, :heads]
    a_d_dst = dstrows[:, 16:16 + heads]
    h_src = srcrows[:, 32:96]
    alpha = a_s_src + a_d_dst
    alpha = jnp.where(alpha >= 0, alpha, 0.2 * alpha)
    ex = jnp.exp(alpha)
    denom = jax.ops.segment_sum(ex, dst, num_segments=num_nodes)
    coef = ex / (denom[dst] + 1e-16)
    ch = h_src.shape[1] // heads
    coef_full = jnp.repeat(coef, ch, axis=1)
    msg = h_src * coef_full
    return jax.ops.segment_sum(msg, dst, num_segments=num_nodes)


def kernel(x, edge_index, emb, W1, att_src1, att_dst1, b1, W2, att_src2,
           att_dst2, b2):
    n = x.shape[0]
    d = emb.shape[1]
    b_pad = ((n + 16 * _NW - 1) // (16 * _NW)) * (16 * _NW)
    idx = jnp.zeros((b_pad,), jnp.int32).at[:n].set(x.astype(jnp.int32))
    emb_pad = jnp.pad(emb, ((0, 0), (0, 128 - d)))
    h0 = _sc_gather_rows(emb_pad, idx, b_pad, b_pad // _NW // 2)[:n, :d]

    loop = jnp.arange(n, dtype=edge_index.dtype)
    src = jnp.concatenate([edge_index[0], loop])
    dst = jnp.concatenate([edge_index[1], loop])
    en = src.shape[0]
    e_pad = ((en + 512 * _NW - 1) // (512 * _NW)) * (512 * _NW)
    src_p = jnp.zeros((e_pad,), jnp.int32).at[:en].set(src.astype(jnp.int32))
    dst_p = jnp.zeros((e_pad,), jnp.int32).at[:en].set(dst.astype(jnp.int32))
    zpad = jnp.zeros((n, 32), jnp.float32)

    def layer(f, W, att_s, att_d, heads, b_prev):
        h, a_s, a_d = _tc_dense(f, W, att_s.reshape(-1), att_d.reshape(-1),
                                heads, b_prev=b_prev)
        packed = jnp.concatenate([a_s, a_d, h, zpad], axis=1)
        srows = _sc_gather_rows(packed, src_p, e_pad, 512)[:en]
        drows = _sc_gather_rows(packed, dst_p, e_pad, 512)[:en]
        return _edge_aggregate(srows, drows, src, dst, heads, n)

    out1 = layer(h0, W1, att_src1, att_dst1, 2, None)
    out2 = layer(out1, W2, att_src2, att_dst2, 1, b1.reshape(1, d))
    return out2 + b2.reshape(1, d)
